# baseline (speedup 1.0000x reference)
import functools

import jax
import jax.numpy as jnp
from jax import lax
from jax.experimental import pallas as pl
from jax.experimental.pallas import tpu as pltpu
from jax.experimental.pallas import tpu_sc as plsc

_N = 512
_D = 128
_K = 10
_T = 2 * _K + 1
_NC = 2
_RPC = _N // _NC
_LO = _N - 11
_S = 64
_RF = _LO + _T
_EARLY = _K + 1


@functools.partial(
    pl.kernel,
    out_type=jax.ShapeDtypeStruct((_N, _N, _D), jnp.float32),
    mesh=plsc.ScalarSubcoreMesh(axis_name="c", num_cores=_NC),
    scratch_types=[
        pltpu.VMEM_SHARED((_RF + 8 * _S, _D), jnp.float32),
        pltpu.SemaphoreType.DMA,
    ],
)
def _sc_band_fill(table_hbm, out_hbm, b_sh, sem):
    cid = lax.axis_index("c")
    r0 = cid * _RPC
    lscr = out_hbm.at[r0 + _EARLY]
    rscr = out_hbm.at[r0 + _EARLY + 1]
    pltpu.sync_copy(table_hbm, b_sh.at[pl.ds(_LO, _T)])
    cs = []
    for k in range(8):
        cs.append(
            pltpu.async_copy(b_sh.at[pl.ds(_LO, 1)], b_sh.at[pl.ds(k, 1)], sem)
        )
        cs.append(
            pltpu.async_copy(
                b_sh.at[pl.ds(_LO + _T - 1, 1)], b_sh.at[pl.ds(_RF + k, 1)], sem
            )
        )
    for c in cs:
        c.wait()
    cs = []
    for k in range(1, 8):
        cs.append(
            pltpu.async_copy(b_sh.at[pl.ds(0, 8)], b_sh.at[pl.ds(8 * k, 8)], sem)
        )
        cs.append(
            pltpu.async_copy(
                b_sh.at[pl.ds(_RF, 8)], b_sh.at[pl.ds(_RF + 8 * k, 8)], sem
            )
        )
    for c in cs:
        c.wait()
    s0 = pltpu.async_copy(b_sh.at[pl.ds(0, _S)], lscr.at[pl.ds(0, _S)], sem)
    s1 = pltpu.async_copy(b_sh.at[pl.ds(_RF, _S)], rscr.at[pl.ds(0, _S)], sem)
    s0.wait()
    s1.wait()

    def read_right_flank():
        return [
            pltpu.async_copy(
                rscr.at[pl.ds(0, _S)], b_sh.at[pl.ds(_RF + _S * k, _S)], sem
            )
            for k in range(1, 8)
        ]

    def read_left_flank():
        cs = [
            pltpu.async_copy(
                lscr.at[pl.ds(0, _S)], b_sh.at[pl.ds(_S * k, _S)], sem
            )
            for k in range(1, 7)
        ]
        cs.append(
            pltpu.async_copy(
                lscr.at[pl.ds(0, 48)], b_sh.at[pl.ds(6 * _S + _S, 48)], sem
            )
        )
        cs.append(
            pltpu.async_copy(b_sh.at[pl.ds(0, 5)], b_sh.at[pl.ds(496, 5)], sem)
        )
        return cs

    def issue_rows(lo, hi):
        def issue(i, carry):
            row = r0 + i
            pltpu.async_copy(
                b_sh.at[pl.ds(_N - 1 - row, _N)], out_hbm.at[row], sem
            )
            return carry

        lax.fori_loop(lo, hi, issue, 0)

    @pl.when(cid == 0)
    def _():
        for c in read_right_flank():
            c.wait()
        issue_rows(0, _EARLY)
        for c in read_left_flank():
            c.wait()
        issue_rows(_EARLY, _RPC)

    @pl.when(cid == 1)
    def _():
        for c in read_left_flank():
            c.wait()
        issue_rows(_RPC - _EARLY, _RPC)
        for c in read_right_flank():
            c.wait()
        issue_rows(0, _RPC - _EARLY)

    def drain(i, carry):
        pltpu.make_async_copy(
            out_hbm.at[0], b_sh.at[pl.ds(0, _N)], sem
        ).wait()
        return carry

    lax.fori_loop(0, _RPC, drain, 0)


def kernel(num_nodes, table):
    del num_nodes
    return _sc_band_fill(table)


# speedup vs baseline: 1.0576x; 1.0576x over previous
import functools

import jax
import jax.numpy as jnp
from jax import lax
from jax.experimental import pallas as pl
from jax.experimental.pallas import tpu as pltpu
from jax.experimental.pallas import tpu_sc as plsc

_N = 512
_D = 128
_K = 10
_T = 2 * _K + 1
_NC = 2
_RPC = _N // _NC
_LO = _N - 11
_S = 64
_RF = _LO + _T
_EARLY = _K + 1


@functools.partial(
    pl.kernel,
    out_type=jax.ShapeDtypeStruct((_N, _N, _D), jnp.float32),
    mesh=plsc.ScalarSubcoreMesh(axis_name="c", num_cores=_NC),
    scratch_types=[
        pltpu.VMEM_SHARED((_RF + 8 + 8 * _S, _D), jnp.float32),
        pltpu.SemaphoreType.DMA,
    ],
)
def _sc_band_fill(table_hbm, out_hbm, b_sh, sem):
    cid = lax.axis_index("c")
    r0 = cid * _RPC
    lscr = out_hbm.at[r0 + _EARLY]
    rscr = out_hbm.at[r0 + _EARLY + 1]
    pltpu.sync_copy(table_hbm, b_sh.at[pl.ds(_LO, _T)])
    cs = []
    for k in range(8):
        cs.append(
            pltpu.async_copy(b_sh.at[pl.ds(_LO, 1)], b_sh.at[pl.ds(k, 1)], sem)
        )
        cs.append(
            pltpu.async_copy(
                b_sh.at[pl.ds(_LO + _T - 1, 1)], b_sh.at[pl.ds(_RF + k, 1)], sem
            )
        )
    for c in cs:
        c.wait()
    cs = []
    for k in range(_S // 8):
        cs.append(
            pltpu.async_copy(b_sh.at[pl.ds(0, 8)], lscr.at[pl.ds(8 * k, 8)], sem)
        )
        cs.append(
            pltpu.async_copy(
                b_sh.at[pl.ds(_RF, 8)], rscr.at[pl.ds(8 * k, 8)], sem
            )
        )
    for c in cs:
        c.wait()

    def read_right_flank():
        cs = [
            pltpu.async_copy(
                rscr.at[pl.ds(0, _S)], b_sh.at[pl.ds(_RF + 8 + _S * k, _S)], sem
            )
            for k in range(8)
        ]
        return cs

    def read_left_flank():
        cs = [
            pltpu.async_copy(
                lscr.at[pl.ds(0, _S)], b_sh.at[pl.ds(8 + _S * k, _S)], sem
            )
            for k in range(7)
        ]
        cs.append(
            pltpu.async_copy(
                lscr.at[pl.ds(0, 32)], b_sh.at[pl.ds(8 + 7 * _S, 32)], sem
            )
        )
        cs.append(
            pltpu.async_copy(b_sh.at[pl.ds(0, 8)], b_sh.at[pl.ds(488, 8)], sem)
        )
        cs.append(
            pltpu.async_copy(b_sh.at[pl.ds(0, 5)], b_sh.at[pl.ds(496, 5)], sem)
        )
        return cs

    def issue_rows(lo, hi):
        def issue(i, carry):
            row = r0 + i
            pltpu.async_copy(
                b_sh.at[pl.ds(_N - 1 - row, _N)], out_hbm.at[row], sem
            )
            return carry

        lax.fori_loop(lo, hi, issue, 0)

    @pl.when(cid == 0)
    def _():
        for c in read_right_flank():
            c.wait()
        issue_rows(0, _EARLY)
        for c in read_left_flank():
            c.wait()
        issue_rows(_EARLY, _RPC)

    @pl.when(cid == 1)
    def _():
        for c in read_left_flank():
            c.wait()
        issue_rows(_RPC - _EARLY, _RPC)
        for c in read_right_flank():
            c.wait()
        issue_rows(0, _RPC - _EARLY)

    def drain(i, carry):
        pltpu.make_async_copy(
            out_hbm.at[0], b_sh.at[pl.ds(0, _N)], sem
        ).wait()
        return carry

    lax.fori_loop(0, _RPC, drain, 0)


def kernel(num_nodes, table):
    del num_nodes
    return _sc_band_fill(table)
